# Initial kernel scaffold; baseline (speedup 1.0000x reference)
import jax
import jax.numpy as jnp
from jax.experimental import pallas as pl


def kernel(data, edge_index, W1, b1, g1, be1, a1, W2, b2, g2, be2, a2):
    raise NotImplementedError("write your pallas kernel here")


# speedup vs baseline: 10.8628x; 10.8628x over previous
import jax
import jax.numpy as jnp
from jax import lax
from jax.experimental import pallas as pl
from jax.experimental.pallas import tpu as pltpu
from jax.experimental.pallas import tpu_sc as plsc

N = 10000
NP = 10240
E = 160000
D = 256
HALF = 128
NC = 2
NS = 16
RPT = NP // NS
K = 100
NCH = (E // NS) // K
KD = 100
NCHD = (E // (NC * NS)) // KD
BM = RPT
GRID = NP // BM
EPS = 1e-5

_F32 = jnp.float32
_MESH_CACHE = []


def _mesh():
    if not _MESH_CACHE:
        _MESH_CACHE.append(plsc.VectorSubcoreMesh(
            core_axis_name="c", subcore_axis_name="s",
            num_cores=NC, num_subcores=NS))
    return _MESH_CACHE[0]


def _deg_sc(dstw, zerosh, onesh):
    @pl.kernel(
        out_type=jax.ShapeDtypeStruct((NC * NP, HALF), _F32),
        mesh=_mesh(),
        scratch_types=[
            pltpu.VMEM((NCHD, KD), jnp.int32),
            pltpu.VMEM((KD, HALF), _F32),
            pltpu.VMEM_SHARED((NP, HALF), _F32),
        ],
    )
    def k(dst_hbm, z_hbm, o_hbm, deg_hbm, idx_v, ones_v, acc):
        c = lax.axis_index("c")
        s = lax.axis_index("s")
        r0 = s * RPT
        pltpu.sync_copy(z_hbm.at[pl.ds(r0, RPT)], acc.at[pl.ds(r0, RPT)])
        pltpu.sync_copy(o_hbm, ones_v)
        pltpu.sync_copy(dst_hbm.at[c * NS + s], idx_v)
        plsc.subcore_barrier()

        @pl.loop(0, NCHD)
        def _(j):
            pltpu.sync_copy(ones_v, acc.at[idx_v.at[j]], add=True)

        plsc.subcore_barrier()
        pltpu.sync_copy(acc.at[pl.ds(r0, RPT)],
                        deg_hbm.at[pl.ds(c * NP + r0, RPT)])

    return k(dstw, zerosh, onesh)


def _scatter_sc(y2, srcg, dst3):
    @pl.kernel(
        out_type=jax.ShapeDtypeStruct((2 * NP, HALF), _F32),
        mesh=_mesh(),
        scratch_types=[
            pltpu.VMEM((NCH, K), jnp.int32),
            pltpu.VMEM((NCH, K), jnp.int32),
            pltpu.VMEM((K, HALF), _F32),
            pltpu.VMEM_SHARED((NP, HALF), _F32),
        ],
    )
    def k(y_hbm, src_hbm, dst_hbm, s_hbm, idxs_v, idxd_v, rows_v, acc):
        c = lax.axis_index("c")
        s = lax.axis_index("s")
        r0 = s * RPT
        g0 = 2 * r0 + c * RPT
        pltpu.sync_copy(src_hbm.at[c * NS + s], idxs_v)
        pltpu.sync_copy(dst_hbm.at[s], idxd_v)
        pltpu.sync_copy(y_hbm.at[pl.ds(g0, RPT)], acc.at[pl.ds(r0, RPT)])
        plsc.subcore_barrier()

        @pl.loop(0, NCH)
        def _(j):
            pltpu.sync_copy(y_hbm.at[idxs_v.at[j]], rows_v)
            pltpu.sync_copy(rows_v, acc.at[idxd_v.at[j]], add=True)

        plsc.subcore_barrier()
        pltpu.sync_copy(acc.at[pl.ds(r0, RPT)], s_hbm.at[pl.ds(g0, RPT)])

    return k(y2, srcg, dst3)


def _dinv_block(d0_ref, d1_ref):
    d = d0_ref[:, 0:1] + d1_ref[:, 0:1] + 1.0
    return lax.rsqrt(d)


def _mm_scale_body(x_ref, w_ref, d0_ref, d1_ref, y_ref):
    xw = lax.dot_general(x_ref[...], w_ref[...], (((1,), (0,)), ((), ())),
                         precision=lax.Precision.HIGHEST,
                         preferred_element_type=_F32)
    y = xw * _dinv_block(d0_ref, d1_ref)
    y_ref[...] = jnp.concatenate([y[:, :HALF], y[:, HALF:]], axis=0)


def _mm_scale(x, w, deg2):
    return pl.pallas_call(
        _mm_scale_body,
        grid=(GRID,),
        in_specs=[
            pl.BlockSpec((BM, D), lambda i: (i, 0)),
            pl.BlockSpec((D, D), lambda i: (0, 0)),
            pl.BlockSpec((BM, HALF), lambda i: (i, 0)),
            pl.BlockSpec((BM, HALF), lambda i: (i + GRID, 0)),
        ],
        out_specs=pl.BlockSpec((2 * BM, HALF), lambda i: (i, 0)),
        out_shape=jax.ShapeDtypeStruct((2 * NP, HALF), _F32),
    )(x, w, deg2, deg2)


def _post_body(slo_ref, shi_ref, d0_ref, d1_ref, b_ref, x_ref, st_ref):
    i = pl.program_id(0)
    x = jnp.concatenate([slo_ref[...], shi_ref[...]], axis=1)
    x = x * _dinv_block(d0_ref, d1_ref) + b_ref[...]
    x_ref[...] = x

    @pl.when(i == 0)
    def _():
        st_ref[...] = jnp.zeros((8, D), _F32)

    rid = lax.broadcasted_iota(jnp.int32, (BM, 1), 0) + i * BM
    xm = jnp.where(rid < N, x, 0.0)
    s1 = jnp.sum(xm, axis=0, keepdims=True)
    s2 = jnp.sum(xm * xm, axis=0, keepdims=True)
    st_ref[...] += jnp.concatenate([s1, s2, jnp.zeros((6, D), _F32)], axis=0)


def _post(s2, deg2, b):
    return pl.pallas_call(
        _post_body,
        grid=(GRID,),
        in_specs=[
            pl.BlockSpec((BM, HALF), lambda i: (2 * i, 0)),
            pl.BlockSpec((BM, HALF), lambda i: (2 * i + 1, 0)),
            pl.BlockSpec((BM, HALF), lambda i: (i, 0)),
            pl.BlockSpec((BM, HALF), lambda i: (i + GRID, 0)),
            pl.BlockSpec((1, D), lambda i: (0, 0)),
        ],
        out_specs=[
            pl.BlockSpec((BM, D), lambda i: (i, 0)),
            pl.BlockSpec((8, D), lambda i: (0, 0)),
        ],
        out_shape=[jax.ShapeDtypeStruct((NP, D), _F32),
                   jax.ShapeDtypeStruct((8, D), _F32)],
    )(s2, s2, deg2, deg2, b)


def _bn_prelu(x_ref, st_ref, g_ref, be_ref, a_ref):
    mean = st_ref[0:1, :] * (1.0 / N)
    var = st_ref[1:2, :] * (1.0 / N) - mean * mean
    scale = g_ref[...] * lax.rsqrt(var + EPS)
    xh = (x_ref[...] - mean) * scale + be_ref[...]
    return jnp.where(xh >= 0, xh, a_ref[...] * xh)


def _bnmm_body(x_ref, st_ref, g_ref, be_ref, a_ref, w_ref, d0_ref, d1_ref,
               y_ref):
    h = _bn_prelu(x_ref, st_ref, g_ref, be_ref, a_ref)
    xw = lax.dot_general(h, w_ref[...], (((1,), (0,)), ((), ())),
                         precision=lax.Precision.HIGHEST,
                         preferred_element_type=_F32)
    y = xw * _dinv_block(d0_ref, d1_ref)
    y_ref[...] = jnp.concatenate([y[:, :HALF], y[:, HALF:]], axis=0)


def _bnmm(x, st, g, be, af, w, deg2):
    return pl.pallas_call(
        _bnmm_body,
        grid=(GRID,),
        in_specs=[
            pl.BlockSpec((BM, D), lambda i: (i, 0)),
            pl.BlockSpec((8, D), lambda i: (0, 0)),
            pl.BlockSpec((1, D), lambda i: (0, 0)),
            pl.BlockSpec((1, D), lambda i: (0, 0)),
            pl.BlockSpec((1, D), lambda i: (0, 0)),
            pl.BlockSpec((D, D), lambda i: (0, 0)),
            pl.BlockSpec((BM, HALF), lambda i: (i, 0)),
            pl.BlockSpec((BM, HALF), lambda i: (i + GRID, 0)),
        ],
        out_specs=pl.BlockSpec((2 * BM, HALF), lambda i: (i, 0)),
        out_shape=jax.ShapeDtypeStruct((2 * NP, HALF), _F32),
    )(x, st, g, be, af, w, deg2, deg2)


def _bnfinal_body(x_ref, st_ref, g_ref, be_ref, a_ref, o_ref):
    o_ref[...] = _bn_prelu(x_ref, st_ref, g_ref, be_ref, a_ref)


def _bnfinal(x, st, g, be, af):
    return pl.pallas_call(
        _bnfinal_body,
        grid=(GRID,),
        in_specs=[
            pl.BlockSpec((BM, D), lambda i: (i, 0)),
            pl.BlockSpec((8, D), lambda i: (0, 0)),
            pl.BlockSpec((1, D), lambda i: (0, 0)),
            pl.BlockSpec((1, D), lambda i: (0, 0)),
            pl.BlockSpec((1, D), lambda i: (0, 0)),
        ],
        out_specs=pl.BlockSpec((BM, D), lambda i: (i, 0)),
        out_shape=jax.ShapeDtypeStruct((NP, D), _F32),
    )(x, st, g, be, af)


def kernel(data, edge_index, W1, b1, g1, be1, a1, W2, b2, g2, be2, a2):
    data_p = jnp.pad(data, ((0, NP - N), (0, 0)))
    src = edge_index[0]
    dst = edge_index[1]
    src_lo = 2 * (src // RPT) * RPT + src % RPT
    srcg = jnp.concatenate([src_lo.reshape(NS, NCH, K),
                            (src_lo + RPT).reshape(NS, NCH, K)], axis=0)
    dst3 = dst.reshape(NS, NCH, K)
    dstw = dst.reshape(NC * NS, NCHD, KD)
    zerosh = jnp.zeros((NP, HALF), _F32)
    onesh = jnp.ones((KD, HALF), _F32)
    b1r = b1.reshape(1, D)
    b2r = b2.reshape(1, D)
    g1r = g1.reshape(1, D)
    g2r = g2.reshape(1, D)
    be1r = be1.reshape(1, D)
    be2r = be2.reshape(1, D)
    a1f = jnp.broadcast_to(a1.reshape(1, 1), (1, D))
    a2f = jnp.broadcast_to(a2.reshape(1, 1), (1, D))

    deg2 = _deg_sc(dstw, zerosh, onesh)
    y1 = _mm_scale(data_p, W1, deg2)
    s1 = _scatter_sc(y1, srcg, dst3)
    x1, st1 = _post(s1, deg2, b1r)
    y2 = _bnmm(x1, st1, g1r, be1r, a1f, W2, deg2)
    s2 = _scatter_sc(y2, srcg, dst3)
    x2, st2 = _post(s2, deg2, b2r)
    return _bnfinal(x2, st2, g2r, be2r, a2f)[:N]


# speedup vs baseline: 13.6314x; 1.2549x over previous
import jax
import jax.numpy as jnp
from jax import lax
from jax.experimental import pallas as pl
from jax.experimental.pallas import tpu as pltpu
from jax.experimental.pallas import tpu_sc as plsc

N = 10000
NP = 10112
E = 160000
D = 256
HALF = 128
NC = 2
NS = 16
RPT = NP // NS
K = 100
NCH = (E // NS) // K
NCHH = NCH // 2
KD = 100
NCHD = (E // (NC * NS)) // KD
BM = RPT
GRID = NP // BM
EPS = 1e-5

_F32 = jnp.float32
_MESH_CACHE = []


def _mesh():
    if not _MESH_CACHE:
        _MESH_CACHE.append(plsc.VectorSubcoreMesh(
            core_axis_name="c", subcore_axis_name="s",
            num_cores=NC, num_subcores=NS))
    return _MESH_CACHE[0]


def _deg_sc(dstw, zerosh, onesh):
    @pl.kernel(
        out_type=jax.ShapeDtypeStruct((NC * NP, HALF), _F32),
        mesh=_mesh(),
        scratch_types=[
            pltpu.VMEM((NCHD, KD), jnp.int32),
            pltpu.VMEM((KD, HALF), _F32),
            pltpu.VMEM_SHARED((NP, HALF), _F32),
        ],
    )
    def k(dst_hbm, z_hbm, o_hbm, deg_hbm, idx_v, ones_v, acc):
        c = lax.axis_index("c")
        s = lax.axis_index("s")
        r0 = s * RPT
        pltpu.sync_copy(z_hbm.at[pl.ds(r0, RPT)], acc.at[pl.ds(r0, RPT)])
        pltpu.sync_copy(o_hbm, ones_v)
        pltpu.sync_copy(dst_hbm.at[c * NS + s], idx_v)
        plsc.subcore_barrier()

        @pl.loop(0, NCHD)
        def _(j):
            pltpu.sync_copy(ones_v, acc.at[idx_v.at[j]], add=True)

        plsc.subcore_barrier()
        pltpu.sync_copy(acc.at[pl.ds(r0, RPT)],
                        deg_hbm.at[pl.ds(c * NP + r0, RPT)])

    return k(dstw, zerosh, onesh)


def _scatter_sc(init2, y2, srcdst):
    @pl.kernel(
        out_type=jax.ShapeDtypeStruct((2 * NP, HALF), _F32),
        mesh=_mesh(),
        scratch_types=[
            pltpu.VMEM((2 * NCHH, K), jnp.int32),
            pltpu.VMEM((2 * K, HALF), _F32),
            pltpu.VMEM_SHARED((NP, HALF), _F32),
            pltpu.SemaphoreType.DMA((2,)),
        ],
    )
    def k(i_hbm, y_hbm, sd_hbm, s_hbm, idx_v, rows, acc, semg):
        c = lax.axis_index("c")
        s = lax.axis_index("s")
        r0 = s * RPT
        g0 = 2 * r0 + c * RPT
        pltpu.sync_copy(sd_hbm.at[c * NS + s], idx_v)
        pltpu.sync_copy(i_hbm.at[pl.ds(g0, RPT)], acc.at[pl.ds(r0, RPT)])
        plsc.subcore_barrier()

        @pl.loop(0, NCHH + 1)
        def _(j):
            @pl.when(j < NCHH)
            def _():
                sl = lax.rem(j, 2)
                pltpu.async_copy(y_hbm.at[idx_v.at[j]],
                                 rows.at[pl.ds(sl * K, K)], semg.at[sl])

            @pl.when(j > 0)
            def _():
                sp = lax.rem(j + 1, 2)
                pltpu.make_async_copy(y_hbm.at[idx_v.at[j - 1]],
                                      rows.at[pl.ds(sp * K, K)],
                                      semg.at[sp]).wait()
                pltpu.sync_copy(rows.at[pl.ds(sp * K, K)],
                                acc.at[idx_v.at[NCHH + j - 1]], add=True)

        plsc.subcore_barrier()
        pltpu.sync_copy(acc.at[pl.ds(r0, RPT)], s_hbm.at[pl.ds(g0, RPT)])

    return k(init2, y2, srcdst)


def _dinv_block(d0_ref, d1_ref):
    d = d0_ref[:, 0:1] + d1_ref[:, 0:1] + 1.0
    return lax.rsqrt(d)


def _mm_scale_body(x_ref, w_ref, d0_ref, d1_ref, y_ref):
    xw = lax.dot_general(x_ref[...], w_ref[...], (((1,), (0,)), ((), ())),
                         precision=lax.Precision.HIGHEST,
                         preferred_element_type=_F32)
    y = xw * _dinv_block(d0_ref, d1_ref)
    y_ref[...] = jnp.concatenate([y[:, :HALF], y[:, HALF:]], axis=0)


def _mm_scale(x, w, deg2):
    return pl.pallas_call(
        _mm_scale_body,
        grid=(GRID,),
        in_specs=[
            pl.BlockSpec((BM, D), lambda i: (i, 0)),
            pl.BlockSpec((D, D), lambda i: (0, 0)),
            pl.BlockSpec((BM, HALF), lambda i: (i, 0)),
            pl.BlockSpec((BM, HALF), lambda i: (i + GRID, 0)),
        ],
        out_specs=pl.BlockSpec((2 * BM, HALF), lambda i: (i, 0)),
        out_shape=jax.ShapeDtypeStruct((2 * NP, HALF), _F32),
    )(x, w, deg2, deg2)


def _post_body(salo_ref, sahi_ref, sblo_ref, sbhi_ref, d0_ref, d1_ref,
               b_ref, x_ref, st_ref):
    i = pl.program_id(0)
    x = jnp.concatenate([salo_ref[...] + sblo_ref[...],
                         sahi_ref[...] + sbhi_ref[...]], axis=1)
    x = x * _dinv_block(d0_ref, d1_ref) + b_ref[...]
    x_ref[...] = x

    @pl.when(i == 0)
    def _():
        st_ref[...] = jnp.zeros((8, D), _F32)

    rid = lax.broadcasted_iota(jnp.int32, (BM, 1), 0) + i * BM
    xm = jnp.where(rid < N, x, 0.0)
    s1 = jnp.sum(xm, axis=0, keepdims=True)
    s2 = jnp.sum(xm * xm, axis=0, keepdims=True)
    st_ref[...] += jnp.concatenate([s1, s2, jnp.zeros((6, D), _F32)], axis=0)


def _post(sa, sb, deg2, b):
    return pl.pallas_call(
        _post_body,
        grid=(GRID,),
        in_specs=[
            pl.BlockSpec((BM, HALF), lambda i: (2 * i, 0)),
            pl.BlockSpec((BM, HALF), lambda i: (2 * i + 1, 0)),
            pl.BlockSpec((BM, HALF), lambda i: (2 * i, 0)),
            pl.BlockSpec((BM, HALF), lambda i: (2 * i + 1, 0)),
            pl.BlockSpec((BM, HALF), lambda i: (i, 0)),
            pl.BlockSpec((BM, HALF), lambda i: (i + GRID, 0)),
            pl.BlockSpec((1, D), lambda i: (0, 0)),
        ],
        out_specs=[
            pl.BlockSpec((BM, D), lambda i: (i, 0)),
            pl.BlockSpec((8, D), lambda i: (0, 0)),
        ],
        out_shape=[jax.ShapeDtypeStruct((NP, D), _F32),
                   jax.ShapeDtypeStruct((8, D), _F32)],
    )(sa, sa, sb, sb, deg2, deg2, b)


def _bn_prelu(x_ref, st_ref, g_ref, be_ref, a_ref):
    mean = st_ref[0:1, :] * (1.0 / N)
    var = st_ref[1:2, :] * (1.0 / N) - mean * mean
    scale = g_ref[...] * lax.rsqrt(var + EPS)
    xh = (x_ref[...] - mean) * scale + be_ref[...]
    return jnp.where(xh >= 0, xh, a_ref[...] * xh)


def _bnmm_body(x_ref, st_ref, g_ref, be_ref, a_ref, w_ref, d0_ref, d1_ref,
               y_ref):
    h = _bn_prelu(x_ref, st_ref, g_ref, be_ref, a_ref)
    xw = lax.dot_general(h, w_ref[...], (((1,), (0,)), ((), ())),
                         precision=lax.Precision.HIGHEST,
                         preferred_element_type=_F32)
    y = xw * _dinv_block(d0_ref, d1_ref)
    y_ref[...] = jnp.concatenate([y[:, :HALF], y[:, HALF:]], axis=0)


def _bnmm(x, st, g, be, af, w, deg2):
    return pl.pallas_call(
        _bnmm_body,
        grid=(GRID,),
        in_specs=[
            pl.BlockSpec((BM, D), lambda i: (i, 0)),
            pl.BlockSpec((8, D), lambda i: (0, 0)),
            pl.BlockSpec((1, D), lambda i: (0, 0)),
            pl.BlockSpec((1, D), lambda i: (0, 0)),
            pl.BlockSpec((1, D), lambda i: (0, 0)),
            pl.BlockSpec((D, D), lambda i: (0, 0)),
            pl.BlockSpec((BM, HALF), lambda i: (i, 0)),
            pl.BlockSpec((BM, HALF), lambda i: (i + GRID, 0)),
        ],
        out_specs=pl.BlockSpec((2 * BM, HALF), lambda i: (i, 0)),
        out_shape=jax.ShapeDtypeStruct((2 * NP, HALF), _F32),
    )(x, st, g, be, af, w, deg2, deg2)


def _bnfinal_body(x_ref, st_ref, g_ref, be_ref, a_ref, o_ref):
    o_ref[...] = _bn_prelu(x_ref, st_ref, g_ref, be_ref, a_ref)


def _bnfinal(x, st, g, be, af):
    return pl.pallas_call(
        _bnfinal_body,
        grid=(GRID,),
        in_specs=[
            pl.BlockSpec((BM, D), lambda i: (i, 0)),
            pl.BlockSpec((8, D), lambda i: (0, 0)),
            pl.BlockSpec((1, D), lambda i: (0, 0)),
            pl.BlockSpec((1, D), lambda i: (0, 0)),
            pl.BlockSpec((1, D), lambda i: (0, 0)),
        ],
        out_specs=pl.BlockSpec((BM, D), lambda i: (i, 0)),
        out_shape=jax.ShapeDtypeStruct((NP, D), _F32),
    )(x, st, g, be, af)


def kernel(data, edge_index, W1, b1, g1, be1, a1, W2, b2, g2, be2, a2):
    data_p = jnp.pad(data, ((0, NP - N), (0, 0)))
    src = edge_index[0]
    dst = edge_index[1]
    src_lo = 2 * (src // RPT) * RPT + src % RPT
    srcg = jnp.concatenate([src_lo.reshape(NS, NCH, K),
                            (src_lo + RPT).reshape(NS, NCH, K)], axis=0)
    dst2 = jnp.concatenate([dst.reshape(NS, NCH, K)] * 2, axis=0)
    srcdst_a = jnp.concatenate([srcg[:, :NCHH], dst2[:, :NCHH]], axis=1)
    srcdst_b = jnp.concatenate([srcg[:, NCHH:], dst2[:, NCHH:]], axis=1)
    dstw = dst.reshape(NC * NS, NCHD, KD)
    zeros2 = jnp.zeros((2 * NP, HALF), _F32)
    zerosh = jnp.zeros((NP, HALF), _F32)
    onesh = jnp.ones((KD, HALF), _F32)
    b1r = b1.reshape(1, D)
    b2r = b2.reshape(1, D)
    g1r = g1.reshape(1, D)
    g2r = g2.reshape(1, D)
    be1r = be1.reshape(1, D)
    be2r = be2.reshape(1, D)
    a1f = jnp.broadcast_to(a1.reshape(1, 1), (1, D))
    a2f = jnp.broadcast_to(a2.reshape(1, 1), (1, D))

    deg2 = _deg_sc(dstw, zerosh, onesh)
    y1 = _mm_scale(data_p, W1, deg2)
    s1a = _scatter_sc(y1, y1, srcdst_a)
    s1b = _scatter_sc(zeros2, y1, srcdst_b)
    x1, st1 = _post(s1a, s1b, deg2, b1r)
    y2 = _bnmm(x1, st1, g1r, be1r, a1f, W2, deg2)
    s2a = _scatter_sc(y2, y2, srcdst_a)
    s2b = _scatter_sc(zeros2, y2, srcdst_b)
    x2, st2 = _post(s2a, s2b, deg2, b2r)
    return _bnfinal(x2, st2, g2r, be2r, a2f)[:N]


# speedup vs baseline: 14.1882x; 1.0408x over previous
import jax
import jax.numpy as jnp
from jax import lax
from jax.experimental import pallas as pl
from jax.experimental.pallas import tpu as pltpu
from jax.experimental.pallas import tpu_sc as plsc

N = 10000
NP = 10112
E = 160000
D = 256
HALF = 128
NC = 2
NS = 16
RPT = NP // NS
K = 100
NCH = (E // NS) // K
NCHH = NCH // 2
KD = 100
NCHD = (E // (NC * NS)) // KD
BM = RPT
GRID = NP // BM
EPS = 1e-5

_F32 = jnp.float32
_MESH_CACHE = []


def _mesh():
    if not _MESH_CACHE:
        _MESH_CACHE.append(plsc.VectorSubcoreMesh(
            core_axis_name="c", subcore_axis_name="s",
            num_cores=NC, num_subcores=NS))
    return _MESH_CACHE[0]


def _deg_sc(dstw, zerosh, onesh):
    @pl.kernel(
        out_type=jax.ShapeDtypeStruct((NC * NP, HALF), _F32),
        mesh=_mesh(),
        scratch_types=[
            pltpu.VMEM((NCHD, KD), jnp.int32),
            pltpu.VMEM((KD, HALF), _F32),
            pltpu.VMEM_SHARED((NP, HALF), _F32),
        ],
    )
    def k(dst_hbm, z_hbm, o_hbm, deg_hbm, idx_v, ones_v, acc):
        c = lax.axis_index("c")
        s = lax.axis_index("s")
        r0 = s * RPT
        pltpu.sync_copy(z_hbm.at[pl.ds(r0, RPT)], acc.at[pl.ds(r0, RPT)])
        pltpu.sync_copy(o_hbm, ones_v)
        pltpu.sync_copy(dst_hbm.at[c * NS + s], idx_v)
        plsc.subcore_barrier()

        @pl.loop(0, NCHD)
        def _(j):
            pltpu.sync_copy(ones_v, acc.at[idx_v.at[j]], add=True)

        plsc.subcore_barrier()
        pltpu.sync_copy(acc.at[pl.ds(r0, RPT)],
                        deg_hbm.at[pl.ds(c * NP + r0, RPT)])

    return k(dstw, zerosh, onesh)


def _scatter_sc(init2, y2, srcdst):
    @pl.kernel(
        out_type=jax.ShapeDtypeStruct((2 * NP, HALF), _F32),
        mesh=_mesh(),
        scratch_types=[
            pltpu.VMEM((2 * NCHH, K), jnp.int32),
            pltpu.VMEM((2 * K, HALF), _F32),
            pltpu.VMEM_SHARED((NP, HALF), _F32),
            pltpu.SemaphoreType.DMA((2,)),
            pltpu.SemaphoreType.DMA((2,)),
            pltpu.SemaphoreType.DMA,
        ],
    )
    def k(i_hbm, y_hbm, sd_hbm, s_hbm, idx_v, rows, acc, semg, sema, semi):
        c = lax.axis_index("c")
        s = lax.axis_index("s")
        r0 = s * RPT
        g0 = 2 * r0 + c * RPT
        init = pltpu.async_copy(i_hbm.at[pl.ds(g0, RPT)],
                                acc.at[pl.ds(r0, RPT)], semi)
        pltpu.sync_copy(sd_hbm.at[c * NS + s], idx_v)
        init.wait()
        plsc.subcore_barrier()

        @pl.loop(0, NCHH + 2)
        def _(j):
            sl = lax.rem(j, 2)

            @pl.when(j >= 2)
            def _():
                pltpu.make_async_copy(rows.at[pl.ds(sl * K, K)],
                                      acc.at[idx_v.at[NCHH + j - 2]],
                                      sema.at[sl]).wait()

            @pl.when(j < NCHH)
            def _():
                pltpu.async_copy(y_hbm.at[idx_v.at[j]],
                                 rows.at[pl.ds(sl * K, K)], semg.at[sl])

            @pl.when((j >= 1) & (j <= NCHH))
            def _():
                sp = lax.rem(j + 1, 2)
                pltpu.make_async_copy(y_hbm.at[idx_v.at[j - 1]],
                                      rows.at[pl.ds(sp * K, K)],
                                      semg.at[sp]).wait()
                pltpu.async_copy(rows.at[pl.ds(sp * K, K)],
                                 acc.at[idx_v.at[NCHH + j - 1]], sema.at[sp],
                                 add=True)

        plsc.subcore_barrier()
        pltpu.sync_copy(acc.at[pl.ds(r0, RPT)], s_hbm.at[pl.ds(g0, RPT)])

    return k(init2, y2, srcdst)


def _dinv_block(d0_ref, d1_ref):
    d = d0_ref[:, 0:1] + d1_ref[:, 0:1] + 1.0
    return lax.rsqrt(d)


def _mm_body(x_ref, w_ref, y_ref):
    xw = lax.dot_general(x_ref[...], w_ref[...], (((1,), (0,)), ((), ())),
                         precision=lax.Precision.HIGHEST,
                         preferred_element_type=_F32)
    y_ref[...] = jnp.concatenate([xw[:, :HALF], xw[:, HALF:]], axis=0)


def _mm(x, w):
    return pl.pallas_call(
        _mm_body,
        grid=(GRID,),
        in_specs=[
            pl.BlockSpec((BM, D), lambda i: (i, 0)),
            pl.BlockSpec((D, D), lambda i: (0, 0)),
        ],
        out_specs=pl.BlockSpec((2 * BM, HALF), lambda i: (i, 0)),
        out_shape=jax.ShapeDtypeStruct((2 * NP, HALF), _F32),
    )(x, w)


def _scale_body(xw_ref, d0_ref, d1_ref, y_ref):
    dinv = _dinv_block(d0_ref, d1_ref)
    y_ref[...] = xw_ref[...] * jnp.concatenate([dinv, dinv], axis=0)


def _scale(xw2, deg2):
    return pl.pallas_call(
        _scale_body,
        grid=(GRID,),
        in_specs=[
            pl.BlockSpec((2 * BM, HALF), lambda i: (i, 0)),
            pl.BlockSpec((BM, HALF), lambda i: (i, 0)),
            pl.BlockSpec((BM, HALF), lambda i: (i + GRID, 0)),
        ],
        out_specs=pl.BlockSpec((2 * BM, HALF), lambda i: (i, 0)),
        out_shape=jax.ShapeDtypeStruct((2 * NP, HALF), _F32),
    )(xw2, deg2, deg2)


def _post_body(salo_ref, sahi_ref, sblo_ref, sbhi_ref, d0_ref, d1_ref,
               b_ref, x_ref, st_ref):
    i = pl.program_id(0)
    x = jnp.concatenate([salo_ref[...] + sblo_ref[...],
                         sahi_ref[...] + sbhi_ref[...]], axis=1)
    x = x * _dinv_block(d0_ref, d1_ref) + b_ref[...]
    x_ref[...] = x

    @pl.when(i == 0)
    def _():
        st_ref[...] = jnp.zeros((8, D), _F32)

    rid = lax.broadcasted_iota(jnp.int32, (BM, 1), 0) + i * BM
    xm = jnp.where(rid < N, x, 0.0)
    s1 = jnp.sum(xm, axis=0, keepdims=True)
    s2 = jnp.sum(xm * xm, axis=0, keepdims=True)
    st_ref[...] += jnp.concatenate([s1, s2, jnp.zeros((6, D), _F32)], axis=0)


def _post(sa, sb, deg2, b):
    return pl.pallas_call(
        _post_body,
        grid=(GRID,),
        in_specs=[
            pl.BlockSpec((BM, HALF), lambda i: (2 * i, 0)),
            pl.BlockSpec((BM, HALF), lambda i: (2 * i + 1, 0)),
            pl.BlockSpec((BM, HALF), lambda i: (2 * i, 0)),
            pl.BlockSpec((BM, HALF), lambda i: (2 * i + 1, 0)),
            pl.BlockSpec((BM, HALF), lambda i: (i, 0)),
            pl.BlockSpec((BM, HALF), lambda i: (i + GRID, 0)),
            pl.BlockSpec((1, D), lambda i: (0, 0)),
        ],
        out_specs=[
            pl.BlockSpec((BM, D), lambda i: (i, 0)),
            pl.BlockSpec((8, D), lambda i: (0, 0)),
        ],
        out_shape=[jax.ShapeDtypeStruct((NP, D), _F32),
                   jax.ShapeDtypeStruct((8, D), _F32)],
    )(sa, sa, sb, sb, deg2, deg2, b)


def _bn_prelu(x_ref, st_ref, g_ref, be_ref, a_ref):
    mean = st_ref[0:1, :] * (1.0 / N)
    var = st_ref[1:2, :] * (1.0 / N) - mean * mean
    scale = g_ref[...] * lax.rsqrt(var + EPS)
    xh = (x_ref[...] - mean) * scale + be_ref[...]
    return jnp.where(xh >= 0, xh, a_ref[...] * xh)


def _bnmm_body(x_ref, st_ref, g_ref, be_ref, a_ref, w_ref, d0_ref, d1_ref,
               y_ref):
    h = _bn_prelu(x_ref, st_ref, g_ref, be_ref, a_ref)
    xw = lax.dot_general(h, w_ref[...], (((1,), (0,)), ((), ())),
                         precision=lax.Precision.HIGHEST,
                         preferred_element_type=_F32)
    y = xw * _dinv_block(d0_ref, d1_ref)
    y_ref[...] = jnp.concatenate([y[:, :HALF], y[:, HALF:]], axis=0)


def _bnmm(x, st, g, be, af, w, deg2):
    return pl.pallas_call(
        _bnmm_body,
        grid=(GRID,),
        in_specs=[
            pl.BlockSpec((BM, D), lambda i: (i, 0)),
            pl.BlockSpec((8, D), lambda i: (0, 0)),
            pl.BlockSpec((1, D), lambda i: (0, 0)),
            pl.BlockSpec((1, D), lambda i: (0, 0)),
            pl.BlockSpec((1, D), lambda i: (0, 0)),
            pl.BlockSpec((D, D), lambda i: (0, 0)),
            pl.BlockSpec((BM, HALF), lambda i: (i, 0)),
            pl.BlockSpec((BM, HALF), lambda i: (i + GRID, 0)),
        ],
        out_specs=pl.BlockSpec((2 * BM, HALF), lambda i: (i, 0)),
        out_shape=jax.ShapeDtypeStruct((2 * NP, HALF), _F32),
    )(x, st, g, be, af, w, deg2, deg2)


def _bnfinal_body(x_ref, st_ref, g_ref, be_ref, a_ref, o_ref):
    o_ref[...] = _bn_prelu(x_ref, st_ref, g_ref, be_ref, a_ref)


def _bnfinal(x, st, g, be, af):
    return pl.pallas_call(
        _bnfinal_body,
        grid=(GRID,),
        in_specs=[
            pl.BlockSpec((BM, D), lambda i: (i, 0)),
            pl.BlockSpec((8, D), lambda i: (0, 0)),
            pl.BlockSpec((1, D), lambda i: (0, 0)),
            pl.BlockSpec((1, D), lambda i: (0, 0)),
            pl.BlockSpec((1, D), lambda i: (0, 0)),
        ],
        out_specs=pl.BlockSpec((BM, D), lambda i: (i, 0)),
        out_shape=jax.ShapeDtypeStruct((N, D), _F32),
    )(x, st, g, be, af)


def kernel(data, edge_index, W1, b1, g1, be1, a1, W2, b2, g2, be2, a2):
    src = edge_index[0]
    dst = edge_index[1]
    src_lo = 2 * (src // RPT) * RPT + src % RPT
    srcg = jnp.concatenate([src_lo.reshape(NS, NCH, K),
                            (src_lo + RPT).reshape(NS, NCH, K)], axis=0)
    dst2 = jnp.concatenate([dst.reshape(NS, NCH, K)] * 2, axis=0)
    srcdst_a = jnp.concatenate([srcg[:, :NCHH], dst2[:, :NCHH]], axis=1)
    srcdst_b = jnp.concatenate([srcg[:, NCHH:], dst2[:, NCHH:]], axis=1)
    dstw = dst.reshape(NC * NS, NCHD, KD)
    zeros2 = jnp.zeros((2 * NP, HALF), _F32)
    zerosh = jnp.zeros((NP, HALF), _F32)
    onesh = jnp.ones((KD, HALF), _F32)
    b1r = b1.reshape(1, D)
    b2r = b2.reshape(1, D)
    g1r = g1.reshape(1, D)
    g2r = g2.reshape(1, D)
    be1r = be1.reshape(1, D)
    be2r = be2.reshape(1, D)
    a1f = jnp.broadcast_to(a1.reshape(1, 1), (1, D))
    a2f = jnp.broadcast_to(a2.reshape(1, 1), (1, D))

    deg2 = _deg_sc(dstw, zerosh, onesh)
    xw1 = _mm(data, W1)
    y1 = _scale(xw1, deg2)
    s1a = _scatter_sc(y1, y1, srcdst_a)
    s1b = _scatter_sc(zeros2, y1, srcdst_b)
    x1, st1 = _post(s1a, s1b, deg2, b1r)
    y2 = _bnmm(x1, st1, g1r, be1r, a1f, W2, deg2)
    s2a = _scatter_sc(y2, y2, srcdst_a)
    s2b = _scatter_sc(zeros2, y2, srcdst_b)
    x2, st2 = _post(s2a, s2b, deg2, b2r)
    return _bnfinal(x2, st2, g2r, be2r, a2f)


# speedup vs baseline: 14.4675x; 1.0197x over previous
import jax
import jax.numpy as jnp
from jax import lax
from jax.experimental import pallas as pl
from jax.experimental.pallas import tpu as pltpu
from jax.experimental.pallas import tpu_sc as plsc

N = 10000
NP = 10112
E = 160000
D = 256
HALF = 128
NC = 2
NS = 16
RPT = NP // NS
K = 100
NCH = (E // NS) // K
NCHH = NCH // 2
KD = 100
NCHD = (E // (NC * NS)) // KD
BM = RPT
GRID = NP // BM
EPS = 1e-5

_F32 = jnp.float32
_MESH_CACHE = []


def _mesh():
    if not _MESH_CACHE:
        _MESH_CACHE.append(plsc.VectorSubcoreMesh(
            core_axis_name="c", subcore_axis_name="s",
            num_cores=NC, num_subcores=NS))
    return _MESH_CACHE[0]


def _deg_sc(dstw, zerosh, onesh):
    @pl.kernel(
        out_type=jax.ShapeDtypeStruct((NC * NP, HALF), _F32),
        mesh=_mesh(),
        scratch_types=[
            pltpu.VMEM((NCHD, KD), jnp.int32),
            pltpu.VMEM((KD, HALF), _F32),
            pltpu.VMEM_SHARED((NP, HALF), _F32),
        ],
    )
    def k(dst_hbm, z_hbm, o_hbm, deg_hbm, idx_v, ones_v, acc):
        c = lax.axis_index("c")
        s = lax.axis_index("s")
        r0 = s * RPT
        pltpu.sync_copy(z_hbm.at[pl.ds(r0, RPT)], acc.at[pl.ds(r0, RPT)])
        pltpu.sync_copy(o_hbm, ones_v)
        pltpu.sync_copy(dst_hbm.at[c * NS + s], idx_v)
        plsc.subcore_barrier()

        @pl.loop(0, NCHD)
        def _(j):
            pltpu.sync_copy(ones_v, acc.at[idx_v.at[j]], add=True)

        plsc.subcore_barrier()
        pltpu.sync_copy(acc.at[pl.ds(r0, RPT)],
                        deg_hbm.at[pl.ds(c * NP + r0, RPT)])

    return k(dstw, zerosh, onesh)


def _scatter_sc(init2, y2, srcdst):
    @pl.kernel(
        out_type=jax.ShapeDtypeStruct((2 * NP, HALF), _F32),
        mesh=_mesh(),
        scratch_types=[
            pltpu.VMEM((2 * NCHH, K), jnp.int32),
            pltpu.VMEM((2 * K, HALF), _F32),
            pltpu.VMEM_SHARED((NP, HALF), _F32),
            pltpu.SemaphoreType.DMA((2,)),
            pltpu.SemaphoreType.DMA((2,)),
            pltpu.SemaphoreType.DMA,
        ],
    )
    def k(i_hbm, y_hbm, sd_hbm, s_hbm, idx_v, rows, acc, semg, sema, semi):
        c = lax.axis_index("c")
        s = lax.axis_index("s")
        r0 = s * RPT
        g0 = 2 * r0 + c * RPT
        init = pltpu.async_copy(i_hbm.at[pl.ds(g0, RPT)],
                                acc.at[pl.ds(r0, RPT)], semi)
        pltpu.sync_copy(sd_hbm.at[c * NS + s], idx_v)
        init.wait()
        plsc.subcore_barrier()

        @pl.loop(0, NCHH + 2)
        def _(j):
            sl = lax.rem(j, 2)

            @pl.when(j >= 2)
            def _():
                pltpu.make_async_copy(rows.at[pl.ds(sl * K, K)],
                                      acc.at[idx_v.at[NCHH + j - 2]],
                                      sema.at[sl]).wait()

            @pl.when(j < NCHH)
            def _():
                pltpu.async_copy(y_hbm.at[idx_v.at[j]],
                                 rows.at[pl.ds(sl * K, K)], semg.at[sl])

            @pl.when((j >= 1) & (j <= NCHH))
            def _():
                sp = lax.rem(j + 1, 2)
                pltpu.make_async_copy(y_hbm.at[idx_v.at[j - 1]],
                                      rows.at[pl.ds(sp * K, K)],
                                      semg.at[sp]).wait()
                pltpu.async_copy(rows.at[pl.ds(sp * K, K)],
                                 acc.at[idx_v.at[NCHH + j - 1]], sema.at[sp],
                                 add=True)

        plsc.subcore_barrier()
        pltpu.sync_copy(acc.at[pl.ds(r0, RPT)], s_hbm.at[pl.ds(g0, RPT)])

    return k(init2, y2, srcdst)


def _dinv_block(d0_ref, d1_ref):
    d = d0_ref[:, 0:1] + d1_ref[:, 0:1] + 1.0
    return lax.rsqrt(d)


def _mm_body(x_ref, w_ref, y_ref):
    xw = lax.dot_general(x_ref[...], w_ref[...], (((1,), (0,)), ((), ())),
                         precision=lax.Precision.HIGHEST,
                         preferred_element_type=_F32)
    y_ref[...] = jnp.concatenate([xw[:, :HALF], xw[:, HALF:]], axis=0)


def _mm(x, w):
    return pl.pallas_call(
        _mm_body,
        grid=(GRID,),
        in_specs=[
            pl.BlockSpec((BM, D), lambda i: (i, 0)),
            pl.BlockSpec((D, D), lambda i: (0, 0)),
        ],
        out_specs=pl.BlockSpec((2 * BM, HALF), lambda i: (i, 0)),
        out_shape=jax.ShapeDtypeStruct((2 * NP, HALF), _F32),
    )(x, w)


def _scale_body(xw_ref, d0_ref, d1_ref, y_ref):
    dinv = _dinv_block(d0_ref, d1_ref)
    y_ref[...] = xw_ref[...] * jnp.concatenate([dinv, dinv], axis=0)


def _scale(xw2, deg2):
    return pl.pallas_call(
        _scale_body,
        grid=(GRID,),
        in_specs=[
            pl.BlockSpec((2 * BM, HALF), lambda i: (i, 0)),
            pl.BlockSpec((BM, HALF), lambda i: (i, 0)),
            pl.BlockSpec((BM, HALF), lambda i: (i + GRID, 0)),
        ],
        out_specs=pl.BlockSpec((2 * BM, HALF), lambda i: (i, 0)),
        out_shape=jax.ShapeDtypeStruct((2 * NP, HALF), _F32),
    )(xw2, deg2, deg2)


def _bn_prelu(x, st_ref, g_ref, be_ref, a_ref):
    mean = st_ref[0:1, :] * (1.0 / N)
    var = st_ref[1:2, :] * (1.0 / N) - mean * mean
    scale = g_ref[...] * lax.rsqrt(var + EPS)
    xh = (x - mean) * scale + be_ref[...]
    return jnp.where(xh >= 0, xh, a_ref[...] * xh)


def _phase1(i, salo_ref, sahi_ref, sblo_ref, sbhi_ref, d0_ref, d1_ref,
            b_ref, xbuf, st_ref):
    x = jnp.concatenate([salo_ref[...] + sblo_ref[...],
                         sahi_ref[...] + sbhi_ref[...]], axis=1)
    x = x * _dinv_block(d0_ref, d1_ref) + b_ref[...]
    xbuf[pl.ds(i * BM, BM), :] = x

    @pl.when(i == 0)
    def _():
        st_ref[...] = jnp.zeros((8, D), _F32)

    rid = lax.broadcasted_iota(jnp.int32, (BM, 1), 0) + i * BM
    xm = jnp.where(rid < N, x, 0.0)
    s1 = jnp.sum(xm, axis=0, keepdims=True)
    s2 = jnp.sum(xm * xm, axis=0, keepdims=True)
    st_ref[...] += jnp.concatenate([s1, s2, jnp.zeros((6, D), _F32)], axis=0)


_IMAP_X = lambda i: (jnp.where(i < GRID, i, i - GRID), 0)
_IMAP_SLO = lambda i: (2 * jnp.where(i < GRID, i, 0), 0)
_IMAP_SHI = lambda i: (2 * jnp.where(i < GRID, i, 0) + 1, 0)
_IMAP_D1 = lambda i: (jnp.where(i < GRID, i, i - GRID) + GRID, 0)
_IMAP_O = lambda i: (jnp.where(i < GRID, 0, i - GRID), 0)
_CONST = lambda i: (0, 0)


def _postbnmm_body(salo_ref, sahi_ref, sblo_ref, sbhi_ref, d0_ref, d1_ref,
                   b_ref, g_ref, be_ref, a_ref, w_ref, y_ref, xbuf, st_ref):
    i = pl.program_id(0)

    @pl.when(i < GRID)
    def _():
        _phase1(i, salo_ref, sahi_ref, sblo_ref, sbhi_ref, d0_ref, d1_ref,
                b_ref, xbuf, st_ref)

    @pl.when(i >= GRID)
    def _():
        ii = i - GRID
        x = xbuf[pl.ds(ii * BM, BM), :]
        h = _bn_prelu(x, st_ref, g_ref, be_ref, a_ref)
        xw = lax.dot_general(h, w_ref[...], (((1,), (0,)), ((), ())),
                             precision=lax.Precision.HIGHEST,
                             preferred_element_type=_F32)
        y = xw * _dinv_block(d0_ref, d1_ref)
        y_ref[...] = jnp.concatenate([y[:, :HALF], y[:, HALF:]], axis=0)


def _postbnmm(sa, sb, deg2, b, g, be, af, w):
    return pl.pallas_call(
        _postbnmm_body,
        grid=(2 * GRID,),
        in_specs=[
            pl.BlockSpec((BM, HALF), _IMAP_SLO),
            pl.BlockSpec((BM, HALF), _IMAP_SHI),
            pl.BlockSpec((BM, HALF), _IMAP_SLO),
            pl.BlockSpec((BM, HALF), _IMAP_SHI),
            pl.BlockSpec((BM, HALF), _IMAP_X),
            pl.BlockSpec((BM, HALF), _IMAP_D1),
            pl.BlockSpec((1, D), _CONST),
            pl.BlockSpec((1, D), _CONST),
            pl.BlockSpec((1, D), _CONST),
            pl.BlockSpec((1, D), _CONST),
            pl.BlockSpec((D, D), _CONST),
        ],
        out_specs=pl.BlockSpec((2 * BM, HALF), _IMAP_O),
        out_shape=jax.ShapeDtypeStruct((2 * NP, HALF), _F32),
        scratch_shapes=[pltpu.VMEM((NP, D), _F32), pltpu.VMEM((8, D), _F32)],
    )(sa, sa, sb, sb, deg2, deg2, b, g, be, af, w)


def _postbnfinal_body(salo_ref, sahi_ref, sblo_ref, sbhi_ref, d0_ref, d1_ref,
                      b_ref, g_ref, be_ref, a_ref, o_ref, xbuf, st_ref):
    i = pl.program_id(0)

    @pl.when(i < GRID)
    def _():
        _phase1(i, salo_ref, sahi_ref, sblo_ref, sbhi_ref, d0_ref, d1_ref,
                b_ref, xbuf, st_ref)

    @pl.when(i >= GRID)
    def _():
        ii = i - GRID
        x = xbuf[pl.ds(ii * BM, BM), :]
        o_ref[...] = _bn_prelu(x, st_ref, g_ref, be_ref, a_ref)


def _postbnfinal(sa, sb, deg2, b, g, be, af):
    return pl.pallas_call(
        _postbnfinal_body,
        grid=(2 * GRID,),
        in_specs=[
            pl.BlockSpec((BM, HALF), _IMAP_SLO),
            pl.BlockSpec((BM, HALF), _IMAP_SHI),
            pl.BlockSpec((BM, HALF), _IMAP_SLO),
            pl.BlockSpec((BM, HALF), _IMAP_SHI),
            pl.BlockSpec((BM, HALF), _IMAP_X),
            pl.BlockSpec((BM, HALF), _IMAP_D1),
            pl.BlockSpec((1, D), _CONST),
            pl.BlockSpec((1, D), _CONST),
            pl.BlockSpec((1, D), _CONST),
            pl.BlockSpec((1, D), _CONST),
        ],
        out_specs=pl.BlockSpec((BM, D), _IMAP_O),
        out_shape=jax.ShapeDtypeStruct((N, D), _F32),
        scratch_shapes=[pltpu.VMEM((NP, D), _F32), pltpu.VMEM((8, D), _F32)],
    )(sa, sa, sb, sb, deg2, deg2, b, g, be, af)


def kernel(data, edge_index, W1, b1, g1, be1, a1, W2, b2, g2, be2, a2):
    src = edge_index[0]
    dst = edge_index[1]
    src_lo = 2 * (src // RPT) * RPT + src % RPT
    srcg = jnp.concatenate([src_lo.reshape(NS, NCH, K),
                            (src_lo + RPT).reshape(NS, NCH, K)], axis=0)
    dst2 = jnp.concatenate([dst.reshape(NS, NCH, K)] * 2, axis=0)
    srcdst_a = jnp.concatenate([srcg[:, :NCHH], dst2[:, :NCHH]], axis=1)
    srcdst_b = jnp.concatenate([srcg[:, NCHH:], dst2[:, NCHH:]], axis=1)
    dstw = dst.reshape(NC * NS, NCHD, KD)
    zeros2 = jnp.zeros((2 * NP, HALF), _F32)
    zerosh = jnp.zeros((NP, HALF), _F32)
    onesh = jnp.ones((KD, HALF), _F32)
    b1r = b1.reshape(1, D)
    b2r = b2.reshape(1, D)
    g1r = g1.reshape(1, D)
    g2r = g2.reshape(1, D)
    be1r = be1.reshape(1, D)
    be2r = be2.reshape(1, D)
    a1f = jnp.broadcast_to(a1.reshape(1, 1), (1, D))
    a2f = jnp.broadcast_to(a2.reshape(1, 1), (1, D))

    deg2 = _deg_sc(dstw, zerosh, onesh)
    xw1 = _mm(data, W1)
    y1 = _scale(xw1, deg2)
    s1a = _scatter_sc(y1, y1, srcdst_a)
    s1b = _scatter_sc(zeros2, y1, srcdst_b)
    y2 = _postbnmm(s1a, s1b, deg2, b1r, g1r, be1r, a1f, W2)
    s2a = _scatter_sc(y2, y2, srcdst_a)
    s2b = _scatter_sc(zeros2, y2, srcdst_b)
    return _postbnfinal(s2a, s2b, deg2, b2r, g2r, be2r, a2f)


# speedup vs baseline: 14.6161x; 1.0103x over previous
import jax
import jax.numpy as jnp
from jax import lax
from jax.experimental import pallas as pl
from jax.experimental.pallas import tpu as pltpu
from jax.experimental.pallas import tpu_sc as plsc

N = 10000
NP = 10112
E = 160000
D = 256
HALF = 128
NC = 2
NS = 16
RPT = NP // NS
K = 100
NCH = (E // NS) // K
NCHH = NCH // 2
KD = 100
NCHD = (E // (NC * NS)) // KD
BM = RPT
GRID = NP // BM
EPS = 1e-5

_F32 = jnp.float32
_MESH_CACHE = []


def _mesh():
    if not _MESH_CACHE:
        _MESH_CACHE.append(plsc.VectorSubcoreMesh(
            core_axis_name="c", subcore_axis_name="s",
            num_cores=NC, num_subcores=NS))
    return _MESH_CACHE[0]


def _deg_sc(dstw, zerosh, onesh):
    @pl.kernel(
        out_type=jax.ShapeDtypeStruct((NC * NP, HALF), _F32),
        mesh=_mesh(),
        scratch_types=[
            pltpu.VMEM((NCHD, KD), jnp.int32),
            pltpu.VMEM((KD, HALF), _F32),
            pltpu.VMEM_SHARED((NP, HALF), _F32),
        ],
    )
    def k(dst_hbm, z_hbm, o_hbm, deg_hbm, idx_v, ones_v, acc):
        c = lax.axis_index("c")
        s = lax.axis_index("s")
        r0 = s * RPT
        pltpu.sync_copy(z_hbm.at[pl.ds(r0, RPT)], acc.at[pl.ds(r0, RPT)])
        pltpu.sync_copy(o_hbm, ones_v)
        pltpu.sync_copy(dst_hbm.at[c * NS + s], idx_v)
        plsc.subcore_barrier()

        @pl.loop(0, NCHD)
        def _(j):
            pltpu.sync_copy(ones_v, acc.at[idx_v.at[j]], add=True)

        plsc.subcore_barrier()
        pltpu.sync_copy(acc.at[pl.ds(r0, RPT)],
                        deg_hbm.at[pl.ds(c * NP + r0, RPT)])

    return k(dstw, zerosh, onesh)


def _scatter_sc(init2, y2, srcdst):
    @pl.kernel(
        out_type=jax.ShapeDtypeStruct((2 * NP, HALF), _F32),
        mesh=_mesh(),
        scratch_types=[
            pltpu.VMEM((2 * NCHH, K), jnp.int32),
            pltpu.VMEM((2 * K, HALF), _F32),
            pltpu.VMEM_SHARED((NP, HALF), _F32),
            pltpu.SemaphoreType.DMA((2,)),
            pltpu.SemaphoreType.DMA((2,)),
            pltpu.SemaphoreType.DMA,
        ],
    )
    def k(i_hbm, y_hbm, sd_hbm, s_hbm, idx_v, rows, acc, semg, sema, semi):
        c = lax.axis_index("c")
        s = lax.axis_index("s")
        r0 = s * RPT
        g0 = 2 * r0 + c * RPT
        init = pltpu.async_copy(i_hbm.at[pl.ds(g0, RPT)],
                                acc.at[pl.ds(r0, RPT)], semi)
        pltpu.sync_copy(sd_hbm.at[c * NS + s], idx_v)
        init.wait()
        plsc.subcore_barrier()

        @pl.loop(0, NCHH + 2)
        def _(j):
            sl = lax.rem(j, 2)

            @pl.when(j >= 2)
            def _():
                pltpu.make_async_copy(rows.at[pl.ds(sl * K, K)],
                                      acc.at[idx_v.at[NCHH + j - 2]],
                                      sema.at[sl]).wait()

            @pl.when(j < NCHH)
            def _():
                pltpu.async_copy(y_hbm.at[idx_v.at[j]],
                                 rows.at[pl.ds(sl * K, K)], semg.at[sl])

            @pl.when((j >= 1) & (j <= NCHH))
            def _():
                sp = lax.rem(j + 1, 2)
                pltpu.make_async_copy(y_hbm.at[idx_v.at[j - 1]],
                                      rows.at[pl.ds(sp * K, K)],
                                      semg.at[sp]).wait()
                pltpu.async_copy(rows.at[pl.ds(sp * K, K)],
                                 acc.at[idx_v.at[NCHH + j - 1]], sema.at[sp],
                                 add=True)

        plsc.subcore_barrier()
        pltpu.sync_copy(acc.at[pl.ds(r0, RPT)], s_hbm.at[pl.ds(g0, RPT)])

    return k(init2, y2, srcdst)


def _dinv_block(d0_ref, d1_ref):
    d = d0_ref[:, 0:1] + d1_ref[:, 0:1] + 1.0
    return lax.rsqrt(d)


def _mm_body(x_ref, w_ref, y_ref):
    xw = lax.dot_general(x_ref[...], w_ref[...], (((1,), (0,)), ((), ())),
                         precision=lax.Precision.DEFAULT,
                         preferred_element_type=_F32)
    y_ref[...] = jnp.concatenate([xw[:, :HALF], xw[:, HALF:]], axis=0)


def _mm(x, w):
    return pl.pallas_call(
        _mm_body,
        grid=(GRID,),
        in_specs=[
            pl.BlockSpec((BM, D), lambda i: (i, 0)),
            pl.BlockSpec((D, D), lambda i: (0, 0)),
        ],
        out_specs=pl.BlockSpec((2 * BM, HALF), lambda i: (i, 0)),
        out_shape=jax.ShapeDtypeStruct((2 * NP, HALF), _F32),
    )(x, w)


def _scale_body(xw_ref, d0_ref, d1_ref, y_ref):
    dinv = _dinv_block(d0_ref, d1_ref)
    y_ref[...] = xw_ref[...] * jnp.concatenate([dinv, dinv], axis=0)


def _scale(xw2, deg2):
    return pl.pallas_call(
        _scale_body,
        grid=(GRID,),
        in_specs=[
            pl.BlockSpec((2 * BM, HALF), lambda i: (i, 0)),
            pl.BlockSpec((BM, HALF), lambda i: (i, 0)),
            pl.BlockSpec((BM, HALF), lambda i: (i + GRID, 0)),
        ],
        out_specs=pl.BlockSpec((2 * BM, HALF), lambda i: (i, 0)),
        out_shape=jax.ShapeDtypeStruct((2 * NP, HALF), _F32),
    )(xw2, deg2, deg2)


def _bn_prelu(x, st_ref, g_ref, be_ref, a_ref):
    mean = st_ref[0:1, :] * (1.0 / N)
    var = st_ref[1:2, :] * (1.0 / N) - mean * mean
    scale = g_ref[...] * lax.rsqrt(var + EPS)
    xh = (x - mean) * scale + be_ref[...]
    return jnp.where(xh >= 0, xh, a_ref[...] * xh)


def _phase1(i, salo_ref, sahi_ref, sblo_ref, sbhi_ref, d0_ref, d1_ref,
            b_ref, xbuf, st_ref):
    x = jnp.concatenate([salo_ref[...] + sblo_ref[...],
                         sahi_ref[...] + sbhi_ref[...]], axis=1)
    x = x * _dinv_block(d0_ref, d1_ref) + b_ref[...]
    xbuf[pl.ds(i * BM, BM), :] = x

    @pl.when(i == 0)
    def _():
        st_ref[...] = jnp.zeros((8, D), _F32)

    rid = lax.broadcasted_iota(jnp.int32, (BM, 1), 0) + i * BM
    xm = jnp.where(rid < N, x, 0.0)
    s1 = jnp.sum(xm, axis=0, keepdims=True)
    s2 = jnp.sum(xm * xm, axis=0, keepdims=True)
    st_ref[...] += jnp.concatenate([s1, s2, jnp.zeros((6, D), _F32)], axis=0)


_IMAP_X = lambda i: (jnp.where(i < GRID, i, i - GRID), 0)
_IMAP_SLO = lambda i: (2 * jnp.where(i < GRID, i, 0), 0)
_IMAP_SHI = lambda i: (2 * jnp.where(i < GRID, i, 0) + 1, 0)
_IMAP_D1 = lambda i: (jnp.where(i < GRID, i, i - GRID) + GRID, 0)
_IMAP_O = lambda i: (jnp.where(i < GRID, 0, i - GRID), 0)
_CONST = lambda i: (0, 0)


def _postbnmm_body(salo_ref, sahi_ref, sblo_ref, sbhi_ref, d0_ref, d1_ref,
                   b_ref, g_ref, be_ref, a_ref, w_ref, y_ref, xbuf, st_ref):
    i = pl.program_id(0)

    @pl.when(i < GRID)
    def _():
        _phase1(i, salo_ref, sahi_ref, sblo_ref, sbhi_ref, d0_ref, d1_ref,
                b_ref, xbuf, st_ref)

    @pl.when(i >= GRID)
    def _():
        ii = i - GRID
        x = xbuf[pl.ds(ii * BM, BM), :]
        h = _bn_prelu(x, st_ref, g_ref, be_ref, a_ref)
        xw = lax.dot_general(h, w_ref[...], (((1,), (0,)), ((), ())),
                             precision=lax.Precision.DEFAULT,
                             preferred_element_type=_F32)
        y = xw * _dinv_block(d0_ref, d1_ref)
        y_ref[...] = jnp.concatenate([y[:, :HALF], y[:, HALF:]], axis=0)


def _postbnmm(sa, sb, deg2, b, g, be, af, w):
    return pl.pallas_call(
        _postbnmm_body,
        grid=(2 * GRID,),
        in_specs=[
            pl.BlockSpec((BM, HALF), _IMAP_SLO),
            pl.BlockSpec((BM, HALF), _IMAP_SHI),
            pl.BlockSpec((BM, HALF), _IMAP_SLO),
            pl.BlockSpec((BM, HALF), _IMAP_SHI),
            pl.BlockSpec((BM, HALF), _IMAP_X),
            pl.BlockSpec((BM, HALF), _IMAP_D1),
            pl.BlockSpec((1, D), _CONST),
            pl.BlockSpec((1, D), _CONST),
            pl.BlockSpec((1, D), _CONST),
            pl.BlockSpec((1, D), _CONST),
            pl.BlockSpec((D, D), _CONST),
        ],
        out_specs=pl.BlockSpec((2 * BM, HALF), _IMAP_O),
        out_shape=jax.ShapeDtypeStruct((2 * NP, HALF), _F32),
        scratch_shapes=[pltpu.VMEM((NP, D), _F32), pltpu.VMEM((8, D), _F32)],
    )(sa, sa, sb, sb, deg2, deg2, b, g, be, af, w)


def _postbnfinal_body(salo_ref, sahi_ref, sblo_ref, sbhi_ref, d0_ref, d1_ref,
                      b_ref, g_ref, be_ref, a_ref, o_ref, xbuf, st_ref):
    i = pl.program_id(0)

    @pl.when(i < GRID)
    def _():
        _phase1(i, salo_ref, sahi_ref, sblo_ref, sbhi_ref, d0_ref, d1_ref,
                b_ref, xbuf, st_ref)

    @pl.when(i >= GRID)
    def _():
        ii = i - GRID
        x = xbuf[pl.ds(ii * BM, BM), :]
        o_ref[...] = _bn_prelu(x, st_ref, g_ref, be_ref, a_ref)


def _postbnfinal(sa, sb, deg2, b, g, be, af):
    return pl.pallas_call(
        _postbnfinal_body,
        grid=(2 * GRID,),
        in_specs=[
            pl.BlockSpec((BM, HALF), _IMAP_SLO),
            pl.BlockSpec((BM, HALF), _IMAP_SHI),
            pl.BlockSpec((BM, HALF), _IMAP_SLO),
            pl.BlockSpec((BM, HALF), _IMAP_SHI),
            pl.BlockSpec((BM, HALF), _IMAP_X),
            pl.BlockSpec((BM, HALF), _IMAP_D1),
            pl.BlockSpec((1, D), _CONST),
            pl.BlockSpec((1, D), _CONST),
            pl.BlockSpec((1, D), _CONST),
            pl.BlockSpec((1, D), _CONST),
        ],
        out_specs=pl.BlockSpec((BM, D), _IMAP_O),
        out_shape=jax.ShapeDtypeStruct((N, D), _F32),
        scratch_shapes=[pltpu.VMEM((NP, D), _F32), pltpu.VMEM((8, D), _F32)],
    )(sa, sa, sb, sb, deg2, deg2, b, g, be, af)


def kernel(data, edge_index, W1, b1, g1, be1, a1, W2, b2, g2, be2, a2):
    src = edge_index[0]
    dst = edge_index[1]
    src_lo = 2 * (src // RPT) * RPT + src % RPT
    srcg = jnp.concatenate([src_lo.reshape(NS, NCH, K),
                            (src_lo + RPT).reshape(NS, NCH, K)], axis=0)
    dst2 = jnp.concatenate([dst.reshape(NS, NCH, K)] * 2, axis=0)
    srcdst_a = jnp.concatenate([srcg[:, :NCHH], dst2[:, :NCHH]], axis=1)
    srcdst_b = jnp.concatenate([srcg[:, NCHH:], dst2[:, NCHH:]], axis=1)
    dstw = dst.reshape(NC * NS, NCHD, KD)
    zeros2 = jnp.zeros((2 * NP, HALF), _F32)
    zerosh = jnp.zeros((NP, HALF), _F32)
    onesh = jnp.ones((KD, HALF), _F32)
    b1r = b1.reshape(1, D)
    b2r = b2.reshape(1, D)
    g1r = g1.reshape(1, D)
    g2r = g2.reshape(1, D)
    be1r = be1.reshape(1, D)
    be2r = be2.reshape(1, D)
    a1f = jnp.broadcast_to(a1.reshape(1, 1), (1, D))
    a2f = jnp.broadcast_to(a2.reshape(1, 1), (1, D))

    deg2 = _deg_sc(dstw, zerosh, onesh)
    xw1 = _mm(data, W1)
    y1 = _scale(xw1, deg2)
    s1a = _scatter_sc(y1, y1, srcdst_a)
    s1b = _scatter_sc(zeros2, y1, srcdst_b)
    y2 = _postbnmm(s1a, s1b, deg2, b1r, g1r, be1r, a1f, W2)
    s2a = _scatter_sc(y2, y2, srcdst_a)
    s2b = _scatter_sc(zeros2, y2, srcdst_b)
    return _postbnfinal(s2a, s2b, deg2, b2r, g2r, be2r, a2f)
